# baseline (speedup 1.0000x reference)
import functools

import jax
import jax.numpy as jnp
from jax import lax
from jax.experimental import pallas as pl
from jax.experimental.pallas import tpu as pltpu
from jax.experimental.pallas import tpu_sc as plsc

_B = 16384
_D = 32
_PACK = 4
_PD = _PACK * _D
_NC = 2
_NS = 16
_L = 16
_NW = _NC * _NS
_BPW = _B // _NW
_CHUNK = 128
_NCHUNK = _BPW // _CHUNK


def _sc_body(table_hbm, idx_hbm, out_hbm, idx_v, idxp_v, rows_v, out_v, w0_v, sem):
    wid = lax.axis_index("s") * _NC + lax.axis_index("c")
    base = wid * _BPW

    pltpu.sync_copy(idx_hbm.at[pl.ds(base, _BPW)], idx_v)
    pltpu.sync_copy(table_hbm.at[pl.ds(0, 1)], w0_v)

    def phys_body(i, carry):
        v = idx_v[pl.ds(i * _L, _L)]
        idxp_v[pl.ds(i * _L, _L)] = lax.shift_right_logical(v, 2)
        return carry

    lax.fori_loop(0, _BPW // _L, phys_body, 0)

    copies = [
        pltpu.async_copy(
            table_hbm.at[idxp_v.at[pl.ds(j * _CHUNK, _CHUNK)]],
            rows_v.at[pl.ds(j * _CHUNK, _CHUNK)],
            sem,
        )
        for j in range(_NCHUNK)
    ]
    for cp in copies:
        cp.wait()

    w0a = w0_v[0, pl.ds(0, _L)]
    w0b = w0_v[0, pl.ds(_L, _L)]

    def group_body(g, carry):
        idx16 = idx_v[pl.ds(g * _L, _L)]
        for j in range(_L):
            r = g * _L + j
            iv = idx16[j]
            off = (iv & 3) * _D
            mv = jnp.full(
                (_L,), jnp.where(iv != 0, jnp.float32(1.0), jnp.float32(0.0))
            )
            va = rows_v[r, pl.ds(off, _L)]
            vb = rows_v[r, pl.ds(off + _L, _L)]
            out_v[r // _PACK, pl.ds((r % _PACK) * _D, _L)] = va * mv + w0a
            out_v[r // _PACK, pl.ds((r % _PACK) * _D + _L, _L)] = vb * mv + w0b
        return carry

    lax.fori_loop(0, _BPW // _L, group_body, 0)

    pltpu.sync_copy(out_v, out_hbm.at[pl.ds(wid * (_BPW // _PACK), _BPW // _PACK)])


@jax.jit
def kernel(user_identifiers, user_embedding_weight):
    table_packed = user_embedding_weight.reshape(-1, _PD)
    mesh = plsc.VectorSubcoreMesh(core_axis_name="c", subcore_axis_name="s")
    run = pl.kernel(
        _sc_body,
        out_type=jax.ShapeDtypeStruct((_B // _PACK, _PD), jnp.float32),
        mesh=mesh,
        scratch_types=[
            pltpu.VMEM((_BPW,), jnp.int32),
            pltpu.VMEM((_BPW,), jnp.int32),
            pltpu.VMEM((_BPW, _PD), jnp.float32),
            pltpu.VMEM((_BPW // _PACK, _PD), jnp.float32),
            pltpu.VMEM((1, _PD), jnp.float32),
            pltpu.SemaphoreType.DMA,
        ],
    )
    out = run(table_packed, user_identifiers)
    return out.reshape(_B, 1, _D)


# speedup vs baseline: 4.7332x; 4.7332x over previous
import functools

import jax
import jax.numpy as jnp
from jax import lax
from jax.experimental import pallas as pl
from jax.experimental.pallas import tpu as pltpu
from jax.experimental.pallas import tpu_sc as plsc

_B = 16384
_D = 32
_V = 1000000
_NC = 2
_NS = 16
_L = 16
_NW = _NC * _NS
_BPW = _B // _NW
_WIN = 128
_K = 8
_NB = _BPW // _K
_MAXBASE = (_V // _WIN - 1) * _WIN
_TAIL = _MAXBASE + _WIN


def _sc_body(
    table_hbm, tail_hbm, idx_hbm, out_hbm, idx_v, win_v, w0_v, tail_v, out_v, sem, semw
):
    wid = lax.axis_index("s") * _NC + lax.axis_index("c")
    base = wid * _BPW

    pltpu.sync_copy(idx_hbm.at[pl.ds(base, _BPW)], idx_v.at[pl.ds(0, _BPW)])
    pltpu.sync_copy(table_hbm.at[:, pl.ds(0, _WIN)], w0_v)
    pltpu.sync_copy(tail_hbm, tail_v)

    lanes = lax.iota(jnp.int32, _L)

    def win_base(i):
        iv = idx_v[pl.ds(i, _L)][0]
        b0 = pl.multiple_of(jnp.minimum((iv >> 7) * _WIN, _MAXBASE), _WIN)
        return b0, iv

    def fire(g, p):
        for k in range(_K):
            b0, _ = win_base(g * _K + k)
            pltpu.async_copy(
                table_hbm.at[:, pl.ds(b0, _WIN)], win_v.at[p, k], sem
            )

    def drain_one(g, k, p):
        b0, iv = win_base(g * _K + k)
        pltpu.make_async_copy(
            table_hbm.at[:, pl.ds(b0, _WIN)], win_v.at[p, k], sem
        ).wait()
        return b0, iv

    w0a = plsc.load_gather(w0_v, [lanes, jnp.zeros((_L,), jnp.int32)])
    w0b = plsc.load_gather(w0_v, [lanes + _L, jnp.zeros((_L,), jnp.int32)])

    fire(0, 0)

    def batch_body(g, carry):
        p = g & 1

        @pl.when(g < _NB - 1)
        def _():
            fire(g + 1, 1 - p)

        for k in range(_K):
            b0, iv = drain_one(g, k, p)
            i = g * _K + k
            col = jnp.minimum(iv - b0, _WIN - 1)
            ps = jnp.full((_L,), p)
            ks = jnp.full((_L,), k)
            cs = jnp.full((_L,), col)
            v0 = plsc.load_gather(win_v, [ps, ks, lanes, cs])
            v1 = plsc.load_gather(win_v, [ps, ks, lanes + _L, cs])
            tcs = jnp.full((_L,), jnp.clip(iv - _TAIL, 0, _V - _TAIL - 1))
            t0 = plsc.load_gather(tail_v, [lanes, tcs])
            t1 = plsc.load_gather(tail_v, [lanes + _L, tcs])
            ts = jnp.full((_L,), iv >= _TAIL)
            v0 = jnp.where(ts, t0, v0)
            v1 = jnp.where(ts, t1, v1)
            ms = jnp.full(
                (_L,), jnp.where(iv != 0, jnp.float32(1.0), jnp.float32(0.0))
            )
            bs = jnp.full((_L,), i)
            plsc.store_scatter(out_v, [lanes, bs], v0 * ms + w0a)
            plsc.store_scatter(out_v, [lanes + _L, bs], v1 * ms + w0b)
        return carry

    lax.fori_loop(0, _NB, batch_body, 0)

    pltpu.sync_copy(out_v, out_hbm.at[:, pl.ds(base, _BPW)])


@jax.jit
def kernel(user_identifiers, user_embedding_weight):
    table_t = user_embedding_weight.T
    tail_t = lax.slice(table_t, (0, _TAIL), (_D, _V))
    mesh = plsc.VectorSubcoreMesh(core_axis_name="c", subcore_axis_name="s")
    run = pl.kernel(
        _sc_body,
        out_type=jax.ShapeDtypeStruct((_D, _B), jnp.float32),
        mesh=mesh,
        scratch_types=[
            pltpu.VMEM((_BPW + _L,), jnp.int32),
            pltpu.VMEM((2, _K, _D, _WIN), jnp.float32),
            pltpu.VMEM((_D, _WIN), jnp.float32),
            pltpu.VMEM((_D, _V - _TAIL), jnp.float32),
            pltpu.VMEM((_D, _BPW), jnp.float32),
            pltpu.SemaphoreType.DMA,
            pltpu.SemaphoreType.DMA,
        ],
        compiler_params=pltpu.CompilerParams(needs_layout_passes=False),
    )
    out = run(table_t, tail_t, user_identifiers)
    return out.T.reshape(_B, 1, _D)
